# baseline (speedup 1.0000x reference)
import jax
import jax.numpy as jnp
from jax.experimental import pallas as pl
from jax.experimental.pallas import tpu as pltpu


def _fused_vgg_kernel(x_ref, w0_ref, w1_ref, b0_ref, b1_ref,
                      o_ref, xh_ref, cat_ref, c1_ref, p_ref):
    Cin = x_ref.shape[1]
    H = x_ref.shape[2]
    W = x_ref.shape[3]
    Hp = H + 2
    C = o_ref.shape[-1]
    M2 = Hp * W
    L = 8

    xh_ref[0:1, 7:9 + W, :] = jnp.zeros((1, W + 2, Cin), jnp.bfloat16)
    xh_ref[Hp - 1:Hp, 7:9 + W, :] = jnp.zeros((1, W + 2, Cin), jnp.bfloat16)
    xh_ref[:, 7:8, :] = jnp.zeros((Hp, 1, Cin), jnp.bfloat16)
    xh_ref[:, 8 + W:9 + W, :] = jnp.zeros((Hp, 1, Cin), jnp.bfloat16)
    t1 = jnp.transpose(x_ref[0].astype(jnp.bfloat16), (1, 0, 2))
    xh_ref[1:H + 1, L:L + W, :] = jnp.transpose(t1, (0, 2, 1))

    cat_ref[:, :, 0:3 * Cin] = jnp.concatenate(
        [xh_ref[:, 7 + dx:7 + dx + W, :] for dx in range(3)], axis=-1)
    cat_ref[:, :, 3 * Cin:2 * C] = jnp.zeros((Hp, W, 2 * C - 3 * Cin),
                                             jnp.bfloat16)

    c1_ref[0:1, 7:9 + W, :] = jnp.zeros((1, W + 2, C), jnp.bfloat16)
    c1_ref[Hp - 1:Hp, 7:9 + W, :] = jnp.zeros((1, W + 2, C), jnp.bfloat16)
    c1_ref[:, 7:8, :] = jnp.zeros((Hp, 1, C), jnp.bfloat16)
    c1_ref[:, 8 + W:9 + W, :] = jnp.zeros((Hp, 1, C), jnp.bfloat16)

    Hh = H // 2
    Mh = (Hh + 2) * W
    for h in range(2):
        r0 = h * Hh
        p_ref[...] = jnp.dot(
            cat_ref[r0:r0 + Hh + 2, :, 0:2 * C].reshape(Mh, 2 * C),
            w0_ref[...],
            preferred_element_type=jnp.float32).reshape(Hh + 2, W, 3 * C)
        a1 = jnp.maximum(
            p_ref[0:Hh, :, 0:C] + p_ref[1:Hh + 1, :, C:2 * C]
            + p_ref[2:Hh + 2, :, 2 * C:] + b0_ref[0],
            0.0).astype(jnp.bfloat16)
        c1_ref[1 + r0:1 + r0 + Hh, L:L + W, :] = a1

    cat_ref[...] = jnp.concatenate(
        [c1_ref[:, 7 + dx:7 + dx + W, :] for dx in range(3)], axis=-1)
    for h in range(2):
        r0 = h * Hh
        p_ref[...] = jnp.dot(
            cat_ref[r0:r0 + Hh + 2].reshape(Mh, 3 * C), w1_ref[...],
            preferred_element_type=jnp.float32).reshape(Hh + 2, W, 3 * C)
        a2 = jnp.maximum(
            p_ref[0:Hh, :, 0:C] + p_ref[1:Hh + 1, :, C:2 * C]
            + p_ref[2:Hh + 2, :, 2 * C:] + b1_ref[0], 0.0)
        r2 = a2.reshape(Hh // 2, 2, W // 2, 2 * C)
        hp = jnp.maximum(r2[:, 0], r2[:, 1])
        o_ref[0, h * (Hh // 2):(h + 1) * (Hh // 2)] = jnp.maximum(
            hp[..., :C], hp[..., C:]).astype(o_ref.dtype)


def kernel(x_nchw, w0, b0, w1, b1):
    N, Cin, H, W = x_nchw.shape
    C = w1.shape[-1]
    Hp = H + 2

    w0r = w0.astype(jnp.bfloat16).reshape(3, 3 * Cin, C)
    w0_all = jnp.concatenate([w0r[0], w0r[1], w0r[2]], axis=-1)
    w0_all = jnp.pad(w0_all, ((0, 2 * C - 3 * Cin), (0, 0)))
    w1r = w1.astype(jnp.bfloat16).reshape(3, 3 * C, C)
    w1_all = jnp.concatenate([w1r[0], w1r[1], w1r[2]], axis=-1)
    b0r = b0.reshape(1, C).astype(jnp.float32)
    b1r = b1.reshape(1, C).astype(jnp.float32)

    y = pl.pallas_call(
        _fused_vgg_kernel,
        out_shape=jax.ShapeDtypeStruct((N, H // 2, W // 2, C), jnp.bfloat16),
        grid=(N,),
        in_specs=[
            pl.BlockSpec((1, Cin, H, W), lambda n: (n, 0, 0, 0)),
            pl.BlockSpec((2 * C, 3 * C), lambda n: (0, 0)),
            pl.BlockSpec((3 * C, 3 * C), lambda n: (0, 0)),
            pl.BlockSpec((1, C), lambda n: (0, 0)),
            pl.BlockSpec((1, C), lambda n: (0, 0)),
        ],
        out_specs=pl.BlockSpec((1, H // 2, W // 2, C), lambda n: (n, 0, 0, 0)),
        scratch_shapes=[
            pltpu.VMEM((Hp, 128, Cin), jnp.bfloat16),
            pltpu.VMEM((Hp, W, 3 * C), jnp.bfloat16),
            pltpu.VMEM((Hp, 128, C), jnp.bfloat16),
            pltpu.VMEM((H // 2 + 2, W, 3 * C), jnp.float32),
        ],
        compiler_params=pltpu.CompilerParams(
            dimension_semantics=("parallel",)),
    )(x_nchw, w0_all, w1_all, b0r, b1r)

    return jnp.transpose(y, (0, 3, 1, 2))


# speedup vs baseline: 1.1773x; 1.1773x over previous
import jax
import jax.numpy as jnp
from jax.experimental import pallas as pl
from jax.experimental.pallas import tpu as pltpu


def _fused_vgg_kernel(x_ref, w0_ref, w1a_ref, w1b_ref, b0_ref, b1_ref,
                      o_ref, xh_ref, cat_ref, c1_ref, p_ref, acc_ref):
    Cin = x_ref.shape[1]
    H = x_ref.shape[2]
    W = x_ref.shape[3]
    Hp = H + 2
    C = o_ref.shape[-1]
    M2 = Hp * W
    M1 = H * W
    L = 8

    xh_ref[0:1, 7:9 + W, :] = jnp.zeros((1, W + 2, Cin), jnp.bfloat16)
    xh_ref[Hp - 1:Hp, 7:9 + W, :] = jnp.zeros((1, W + 2, Cin), jnp.bfloat16)
    xh_ref[:, 7:8, :] = jnp.zeros((Hp, 1, Cin), jnp.bfloat16)
    xh_ref[:, 8 + W:9 + W, :] = jnp.zeros((Hp, 1, Cin), jnp.bfloat16)
    t1 = jnp.transpose(x_ref[0].astype(jnp.bfloat16), (1, 0, 2))
    xh_ref[1:H + 1, L:L + W, :] = jnp.transpose(t1, (0, 2, 1))

    cat_ref[:, :, 0:3 * Cin] = jnp.concatenate(
        [xh_ref[:, 7 + dx:7 + dx + W, :] for dx in range(3)], axis=-1)
    cat_ref[:, :, 3 * Cin:] = jnp.zeros((Hp, W, 2 * C - 3 * Cin),
                                        jnp.bfloat16)
    p_ref[...] = jnp.dot(
        cat_ref[...].reshape(M2, 2 * C), w0_ref[...],
        preferred_element_type=jnp.float32).reshape(Hp, W, 3 * C)
    acc_ref[...] = (p_ref[0:H, :, 0:C] + p_ref[1:H + 1, :, C:2 * C]
                    + p_ref[2:H + 2, :, 2 * C:]).reshape(M1, C)

    a1 = jnp.maximum(acc_ref[...] + b0_ref[...], 0.0).astype(jnp.bfloat16)
    c1_ref[0:1, 7:9 + W, :] = jnp.zeros((1, W + 2, C), jnp.bfloat16)
    c1_ref[Hp - 1:Hp, 7:9 + W, :] = jnp.zeros((1, W + 2, C), jnp.bfloat16)
    c1_ref[:, 7:8, :] = jnp.zeros((Hp, 1, C), jnp.bfloat16)
    c1_ref[:, 8 + W:9 + W, :] = jnp.zeros((Hp, 1, C), jnp.bfloat16)
    c1_ref[1:H + 1, L:L + W, :] = a1.reshape(H, W, C)

    cat_ref[...] = jnp.concatenate(
        [c1_ref[:, 7:7 + W, :], c1_ref[:, 8:8 + W, :]], axis=-1)
    p_ref[...] = jnp.dot(
        cat_ref[...].reshape(M2, 2 * C), w1a_ref[...],
        preferred_element_type=jnp.float32).reshape(Hp, W, 3 * C)
    acc_ref[...] = (p_ref[0:H, :, 0:C] + p_ref[1:H + 1, :, C:2 * C]
                    + p_ref[2:H + 2, :, 2 * C:]).reshape(M1, C)
    p_ref[...] = jnp.dot(
        c1_ref[:, 9:9 + W, :].reshape(M2, C), w1b_ref[...],
        preferred_element_type=jnp.float32).reshape(Hp, W, 3 * C)
    acc_ref[...] += (p_ref[0:H, :, 0:C] + p_ref[1:H + 1, :, C:2 * C]
                     + p_ref[2:H + 2, :, 2 * C:]).reshape(M1, C)

    a2 = jnp.maximum(acc_ref[...] + b1_ref[...], 0.0)
    r2 = a2.reshape(H // 2, 2, W // 2, 2 * C)
    hp = jnp.maximum(r2[:, 0], r2[:, 1])
    o_ref[0] = jnp.maximum(hp[..., :C], hp[..., C:]).astype(o_ref.dtype)


def kernel(x_nchw, w0, b0, w1, b1):
    N, Cin, H, W = x_nchw.shape
    C = w1.shape[-1]
    Hp = H + 2

    w0r = w0.astype(jnp.bfloat16).reshape(3, 3 * Cin, C)
    w0_all = jnp.concatenate([w0r[0], w0r[1], w0r[2]], axis=-1)
    w0_all = jnp.pad(w0_all, ((0, 2 * C - 3 * Cin), (0, 0)))
    w1r = w1[:, 0:2].astype(jnp.bfloat16).reshape(3, 2 * C, C)
    w1a = jnp.concatenate([w1r[0], w1r[1], w1r[2]], axis=-1)
    w1s = w1[:, 2].astype(jnp.bfloat16)
    w1b = jnp.concatenate([w1s[0], w1s[1], w1s[2]], axis=-1)
    b0r = b0.reshape(1, C).astype(jnp.float32)
    b1r = b1.reshape(1, C).astype(jnp.float32)

    y = pl.pallas_call(
        _fused_vgg_kernel,
        out_shape=jax.ShapeDtypeStruct((N, H // 2, W // 2, C), jnp.bfloat16),
        grid=(N,),
        in_specs=[
            pl.BlockSpec((1, Cin, H, W), lambda n: (n, 0, 0, 0)),
            pl.BlockSpec((2 * C, 3 * C), lambda n: (0, 0)),
            pl.BlockSpec((2 * C, 3 * C), lambda n: (0, 0)),
            pl.BlockSpec((C, 3 * C), lambda n: (0, 0)),
            pl.BlockSpec((1, C), lambda n: (0, 0)),
            pl.BlockSpec((1, C), lambda n: (0, 0)),
        ],
        out_specs=pl.BlockSpec((1, H // 2, W // 2, C), lambda n: (n, 0, 0, 0)),
        scratch_shapes=[
            pltpu.VMEM((Hp, 128, Cin), jnp.bfloat16),
            pltpu.VMEM((Hp, W, 2 * C), jnp.bfloat16),
            pltpu.VMEM((Hp, 128, C), jnp.bfloat16),
            pltpu.VMEM((Hp, W, 3 * C), jnp.float32),
            pltpu.VMEM((H * W, C), jnp.float32),
        ],
        compiler_params=pltpu.CompilerParams(
            dimension_semantics=("parallel",)),
    )(x_nchw, w0_all, w1a, w1b, b0r, b1r)

    return jnp.transpose(y, (0, 3, 1, 2))
